# baseline (speedup 1.0000x reference)
import functools

import jax
import jax.numpy as jnp
from jax import lax
from jax.experimental import pallas as pl
from jax.experimental.pallas import tpu as pltpu
from jax.experimental.pallas import tpu_sc as plsc

N, C, H, W = 32, 3, 512, 512
GRID = 4
PH, PW = H // GRID, W // GRID
NCHUNKS = GRID * GRID
NBUF = 2
NROUNDS = NCHUNKS // NBUF


def _sc_body(img_hbm, scores_hbm, out_hbm, scores_v, buf_v, gsem, wsem):
    wid = lax.axis_index("s") * 2 + lax.axis_index("c")
    lanes = lax.iota(jnp.int32, 16)

    pltpu.sync_copy(scores_hbm.at[wid], scores_v)
    s = scores_v[...]
    rank = lanes & 0
    for t in range(16):
        st = s[t]
        cond = (st < s) | ((st == s) & (t < lanes))
        rank = rank + jnp.where(cond, 1, 0)

    def gather(k, b):
        p = k
        q = jnp.sum(jnp.where(rank == p, lanes, 0))
        qr = q >> 2
        qc = q & 3
        return pltpu.make_async_copy(
            img_hbm.at[wid, :, pl.ds(qr * PH, PH), pl.ds(qc * PW, PW)],
            buf_v.at[b], gsem.at[b])

    def writeback(k, b):
        p = k
        r = p >> 2
        cb = p & 3
        return pltpu.make_async_copy(
            buf_v.at[b],
            out_hbm.at[wid, :, pl.ds(r * PH, PH), pl.ds(cb * PW, PW)],
            wsem.at[b])

    for b in range(NBUF):
        gather(b, b).start()

    def round_body(k2, carry):
        k0 = k2 * NBUF
        for b in range(NBUF):
            gather(k0 + b, b).wait()
            writeback(k0 + b, b).start()
        @pl.when(k2 < NROUNDS - 1)
        def _():
            for b in range(NBUF):
                writeback(k0 + b, b).wait()
                gather(k0 + NBUF + b, b).start()
        return carry

    lax.fori_loop(0, NROUNDS, round_body, 0)

    for b in range(NBUF):
        writeback(NCHUNKS - NBUF + b, b).wait()


@jax.jit
def kernel(img):
    assert img.shape == (N, C, H, W)
    pkey = jax.random.key(42)
    scores = jax.random.uniform(pkey, (N, GRID * GRID), dtype=jnp.float32)

    mesh = plsc.VectorSubcoreMesh(core_axis_name="c", subcore_axis_name="s")
    run = functools.partial(
        pl.kernel,
        mesh=mesh,
        out_type=jax.ShapeDtypeStruct((N, C, H, W), jnp.float32),
        scratch_types=[
            pltpu.VMEM((16,), jnp.float32),
            pltpu.VMEM((NBUF, C, PH, PW), jnp.float32),
            pltpu.SemaphoreType.DMA((NBUF,)),
            pltpu.SemaphoreType.DMA((NBUF,)),
        ],
        compiler_params=pltpu.CompilerParams(needs_layout_passes=False),
    )(_sc_body)
    return run(img, scores)


# speedup vs baseline: 1.0802x; 1.0802x over previous
import functools

import jax
import jax.numpy as jnp
from jax import lax
from jax.experimental import pallas as pl
from jax.experimental.pallas import tpu as pltpu
from jax.experimental.pallas import tpu_sc as plsc

N, C, H, W = 32, 3, 512, 512
GRID = 4
PH, PW = H // GRID, W // GRID
NCHUNKS = C * GRID * GRID
NBUF = 6
NROUNDS = NCHUNKS // NBUF


def _sc_body(img_hbm, scores_hbm, out_hbm, scores_v, perm_s, buf_v, gsem, wsem):
    wid = lax.axis_index("s") * 2 + lax.axis_index("c")
    lanes = lax.iota(jnp.int32, 16)

    pltpu.sync_copy(scores_hbm.at[wid], scores_v)
    s = scores_v[...]
    rank = lanes & 0
    for t in range(16):
        st = s[t]
        cond = (st < s) | ((st == s) & (t < lanes))
        rank = rank + jnp.where(cond, 1, 0)
    for t in range(16):
        perm_s[rank[t]] = t

    def gather(k, b):
        c = k >> 4
        p = k & 15
        q = perm_s[p]
        qr = q >> 2
        qc = q & 3
        return pltpu.make_async_copy(
            img_hbm.at[wid, c, pl.ds(qr * PH, PH), pl.ds(qc * PW, PW)],
            buf_v.at[b], gsem.at[b])

    def writeback(k, b):
        c = k >> 4
        p = k & 15
        r = p >> 2
        cb = p & 3
        return pltpu.make_async_copy(
            buf_v.at[b],
            out_hbm.at[wid, c, pl.ds(r * PH, PH), pl.ds(cb * PW, PW)],
            wsem.at[b])

    def gather_wait(b):
        pltpu.make_async_copy(
            img_hbm.at[wid, 0, pl.ds(0, PH), pl.ds(0, PW)],
            buf_v.at[b], gsem.at[b]).wait()

    def writeback_wait(b):
        pltpu.make_async_copy(
            buf_v.at[b],
            out_hbm.at[wid, 0, pl.ds(0, PH), pl.ds(0, PW)],
            wsem.at[b]).wait()

    for b in range(NBUF):
        gather(b, b).start()

    def round_body(k2, carry):
        k0 = k2 * NBUF
        for b in range(NBUF):
            gather_wait(b)
            writeback(k0 + b, b).start()
        @pl.when(k2 < NROUNDS - 1)
        def _():
            for b in range(NBUF):
                writeback_wait(b)
                gather(k0 + NBUF + b, b).start()
        return carry

    lax.fori_loop(0, NROUNDS, round_body, 0)

    for b in range(NBUF):
        writeback_wait(b)


@jax.jit
def kernel(img):
    assert img.shape == (N, C, H, W)
    pkey = jax.random.key(42)
    scores = jax.random.uniform(pkey, (N, GRID * GRID), dtype=jnp.float32)

    mesh = plsc.VectorSubcoreMesh(core_axis_name="c", subcore_axis_name="s")
    run = functools.partial(
        pl.kernel,
        mesh=mesh,
        out_type=jax.ShapeDtypeStruct((N, C, H, W), jnp.float32),
        scratch_types=[
            pltpu.VMEM((16,), jnp.float32),
            pltpu.SMEM((16,), jnp.int32),
            pltpu.VMEM((NBUF, PH, PW), jnp.float32),
            pltpu.SemaphoreType.DMA((NBUF,)),
            pltpu.SemaphoreType.DMA((NBUF,)),
        ],
        compiler_params=pltpu.CompilerParams(needs_layout_passes=False),
    )(_sc_body)
    return run(img, scores)


# speedup vs baseline: 1.0898x; 1.0089x over previous
import functools

import jax
import jax.numpy as jnp
from jax import lax
from jax.experimental import pallas as pl
from jax.experimental.pallas import tpu as pltpu
from jax.experimental.pallas import tpu_sc as plsc

N, C, H, W = 32, 3, 512, 512
GRID = 4
PH, PW = H // GRID, W // GRID
HH = PH // 2
NCHUNKS = C * GRID * GRID * 2
NBUF = 12
NROUNDS = NCHUNKS // NBUF


def _sc_body(img_hbm, scores_hbm, out_hbm, scores_v, perm_s, buf_v, gsem, wsem):
    wid = lax.axis_index("s") * 2 + lax.axis_index("c")
    lanes = lax.iota(jnp.int32, 16)

    pltpu.sync_copy(scores_hbm.at[wid], scores_v)
    s = scores_v[...]
    rank = lanes & 0
    for t in range(16):
        st = s[t]
        cond = (st < s) | ((st == s) & (t < lanes))
        rank = rank + jnp.where(cond, 1, 0)
    for t in range(16):
        perm_s[rank[t]] = t

    def gather(k, b):
        c = k >> 5
        p = (k >> 1) & 15
        h = k & 1
        q = perm_s[p]
        qr = q >> 2
        qc = q & 3
        return pltpu.make_async_copy(
            img_hbm.at[wid, c, pl.ds(qr * PH + h * HH, HH), pl.ds(qc * PW, PW)],
            buf_v.at[b], gsem.at[b])

    def writeback(k, b):
        c = k >> 5
        p = (k >> 1) & 15
        h = k & 1
        r = p >> 2
        cb = p & 3
        return pltpu.make_async_copy(
            buf_v.at[b],
            out_hbm.at[wid, c, pl.ds(r * PH + h * HH, HH), pl.ds(cb * PW, PW)],
            wsem.at[b])

    def gather_wait(b):
        pltpu.make_async_copy(
            img_hbm.at[wid, 0, pl.ds(0, HH), pl.ds(0, PW)],
            buf_v.at[b], gsem.at[b]).wait()

    def writeback_wait(b):
        pltpu.make_async_copy(
            buf_v.at[b],
            out_hbm.at[wid, 0, pl.ds(0, HH), pl.ds(0, PW)],
            wsem.at[b]).wait()

    for b in range(NBUF):
        gather(b, b).start()

    def round_body(k2, carry):
        k0 = k2 * NBUF
        for b in range(NBUF):
            gather_wait(b)
            writeback(k0 + b, b).start()
        @pl.when(k2 < NROUNDS - 1)
        def _():
            for b in range(NBUF):
                writeback_wait(b)
                gather(k0 + NBUF + b, b).start()
        return carry

    lax.fori_loop(0, NROUNDS, round_body, 0)

    for b in range(NBUF):
        writeback_wait(b)


@jax.jit
def kernel(img):
    assert img.shape == (N, C, H, W)
    pkey = jax.random.key(42)
    scores = jax.random.uniform(pkey, (N, GRID * GRID), dtype=jnp.float32)

    mesh = plsc.VectorSubcoreMesh(core_axis_name="c", subcore_axis_name="s")
    run = functools.partial(
        pl.kernel,
        mesh=mesh,
        out_type=jax.ShapeDtypeStruct((N, C, H, W), jnp.float32),
        scratch_types=[
            pltpu.VMEM((16,), jnp.float32),
            pltpu.SMEM((16,), jnp.int32),
            pltpu.VMEM((NBUF, HH, PW), jnp.float32),
            pltpu.SemaphoreType.DMA((NBUF,)),
            pltpu.SemaphoreType.DMA((NBUF,)),
        ],
        compiler_params=pltpu.CompilerParams(needs_layout_passes=False),
    )(_sc_body)
    return run(img, scores)
